# baseline (speedup 1.0000x reference)
import jax
import jax.numpy as jnp
from jax import lax
from jax.experimental import pallas as pl
from jax.experimental.pallas import tpu as pltpu
from jax.experimental.pallas import tpu_sc as plsc

N = 10000
E = 320000
U = 128
U2 = U // 2

NC = 2
NS = 16
CH = 128
NCH = 168
EPT = CH * NCH
EPAD = NS * EPT
NROWS = N + 8
ZROWS = 200
NROWCH = N // ZROWS
MAXJ = (NROWCH + NS - 1) // NS


def _prep_body(h_ref, wo_ref, bo_ref, wa_ref, ba_ref, hgeo_ref, hsplit_ref,
               hadj_ref):
    h = h_ref[...]
    dn = (((1,), (1,)), ((), ()))
    hgeo = lax.dot_general(h, wo_ref[...], dn,
                           preferred_element_type=jnp.float32) + bo_ref[...]
    hgeo_ref[...] = hgeo
    hsplit_ref[0] = hgeo[:, :U2]
    hsplit_ref[1] = hgeo[:, U2:]
    hadj_ref[...] = lax.dot_general(h, wa_ref[...], dn,
                                    preferred_element_type=jnp.float32) + ba_ref[...]


def _sc_body(hsplit_hbm, pk_hbm, outn_hbm, outd_hbm,
             pk0, pk1, pk2, pk3, rows0, rows1, rows2, ones_v, zeros_v,
             zeros16_v, neigh_sp, deg_sp,
             semi0, semi1, semi2, semi3, semg0, semg1, semg2,
             sems0, sems1, sems2):
    cid = lax.axis_index("c")
    sid = lax.axis_index("s")

    zvec = jnp.zeros((16,), jnp.float32)
    onehot = jnp.where(lax.iota(jnp.int32, 16) == 0, 1.0, 0.0).astype(jnp.float32)

    def _fill_zeros(i, carry):
        r = i // (U2 // 16)
        f = i % (U2 // 16)
        zeros_v[r, pl.ds(f * 16, 16)] = zvec
        return carry
    lax.fori_loop(0, ZROWS * (U2 // 16), _fill_zeros, 0)

    def _fill_zeros16(i, carry):
        zeros16_v[i, :] = zvec
        return carry
    lax.fori_loop(0, ZROWS, _fill_zeros16, 0)

    def _fill_ones(i, carry):
        ones_v[i, :] = onehot
        return carry
    lax.fori_loop(0, CH, _fill_ones, 0)

    for j in range(MAXJ):
        k = sid + NS * j

        @pl.when(k < NROWCH)
        def _():
            pltpu.sync_copy(zeros_v, neigh_sp.at[pl.ds(k * ZROWS, ZROWS)])
            pltpu.sync_copy(zeros16_v, deg_sp.at[pl.ds(k * ZROWS, ZROWS)])
    plsc.subcore_barrier()

    table = hsplit_hbm.at[cid]
    me = pk_hbm.at[sid]

    pks = (pk0, pk1, pk2, pk3)
    rowss = (rows0, rows1, rows2)
    semis = (semi0, semi1, semi2, semi3)
    semgs = (semg0, semg1, semg2)
    semss = (sems0, sems1, sems2)

    def _iter(i, b):
        pk_i = pks[b % 4]
        pk_n1 = pks[(b + 1) % 4]
        pk_n2 = pks[(b + 2) % 4]
        pk_p2 = pks[(b + 2) % 4]
        rows_i = rowss[b % 3]
        rows_n1 = rowss[(b + 1) % 3]
        rows_p2 = rowss[(b + 1) % 3]

        @pl.when(i > 1)
        def _():
            pltpu.make_async_copy(rows_p2, neigh_sp.at[pk_p2.at[1]],
                                  semss[(b + 1) % 3]).wait()
            pltpu.make_async_copy(ones_v, deg_sp.at[pk_p2.at[1]],
                                  semss[(b + 1) % 3]).wait()

        @pl.when(i + 1 < NCH)
        def _():
            pltpu.make_async_copy(me.at[i + 1], pk_n1,
                                  semis[(b + 1) % 4]).wait()
            pltpu.async_copy(table.at[pk_n1.at[0]], rows_n1,
                             semgs[(b + 1) % 3])

        @pl.when(i + 2 < NCH)
        def _():
            pltpu.async_copy(me.at[i + 2], pk_n2, semis[(b + 2) % 4])

        pltpu.make_async_copy(table.at[pk_i.at[0]], rows_i,
                              semgs[b % 3]).wait()

        @plsc.parallel_loop(0, CH, 1, unroll=4)
        def _mul(e):
            efv = plsc.bitcast(
                plsc.load_gather(pk_i, [jnp.full((16,), 2, jnp.int32),
                                        jnp.full((16,), e, jnp.int32)]),
                jnp.float32)
            for f in range(U2 // 16):
                sl = pl.ds(f * 16, 16)
                rows_i[e, sl] = rows_i[e, sl] * efv

        pltpu.make_async_copy(rows_i, neigh_sp.at[pk_i.at[1]],
                              semss[b % 3]).start(add=True)
        pltpu.make_async_copy(ones_v, deg_sp.at[pk_i.at[1]],
                              semss[b % 3]).start(add=True)

    pltpu.async_copy(me.at[0], pk0, semi0)
    pltpu.make_async_copy(me.at[0], pk0, semi0).wait()
    pltpu.async_copy(table.at[pk0.at[0]], rows0, semg0)
    pltpu.async_copy(me.at[1], pk1, semi1)

    @pl.loop(0, NCH, step=12)
    def _(i):
        for b in range(12):
            _iter(i + b, b)

    pltpu.make_async_copy(rowss[(NCH - 2) % 3],
                          neigh_sp.at[pks[(NCH - 2) % 4].at[1]],
                          semss[(NCH - 2) % 3]).wait()
    pltpu.make_async_copy(ones_v, deg_sp.at[pks[(NCH - 2) % 4].at[1]],
                          semss[(NCH - 2) % 3]).wait()
    pltpu.make_async_copy(rowss[(NCH - 1) % 3],
                          neigh_sp.at[pks[(NCH - 1) % 4].at[1]],
                          semss[(NCH - 1) % 3]).wait()
    pltpu.make_async_copy(ones_v, deg_sp.at[pks[(NCH - 1) % 4].at[1]],
                          semss[(NCH - 1) % 3]).wait()

    plsc.subcore_barrier()

    for j in range(MAXJ):
        k = sid + NS * j

        @pl.when(k < NROWCH)
        def _():
            r0 = k * ZROWS
            pltpu.sync_copy(neigh_sp.at[pl.ds(r0, ZROWS)], zeros_v)
            pltpu.sync_copy(zeros_v, outn_hbm.at[cid, pl.ds(r0, ZROWS)])
            pltpu.sync_copy(deg_sp.at[pl.ds(r0, ZROWS)], zeros16_v)
            pltpu.sync_copy(zeros16_v, outd_hbm.at[cid, pl.ds(r0, ZROWS)])


BM = 400
GM = N // BM


def _mm_body(ag_ref, hadj_ref, acc_ref):
    m = pl.program_id(0)

    @pl.when(m == 0)
    def _():
        acc_ref[...] = jnp.zeros_like(acc_ref)

    acc_ref[...] += lax.dot_general(
        ag_ref[...], hadj_ref[...], (((0,), (0,)), ((), ())),
        preferred_element_type=jnp.float32)


def _epi_body(acc_ref, hgeo_ref, pn_ref, pd_ref, ws_ref, wn_ref, bs_ref,
              out_ref):
    deg = pd_ref[0, :, 0:1]
    neigh = jnp.concatenate([pn_ref[0], pn_ref[1]], axis=-1)
    neigh = neigh / jnp.maximum(deg, 1.0)
    dn = (((1,), (1,)), ((), ()))
    geo = (lax.dot_general(hgeo_ref[...], ws_ref[...], dn,
                           preferred_element_type=jnp.float32)
           + lax.dot_general(neigh, wn_ref[...], dn,
                             preferred_element_type=jnp.float32)
           + bs_ref[...])
    tot = geo + acc_ref[...]
    mean = jnp.mean(tot, axis=-1, keepdims=True)
    var = jnp.mean((tot - mean) * (tot - mean), axis=-1, keepdims=True)
    xn = (tot - mean) * lax.rsqrt(var + 1e-5)
    out_ref[...] = jnp.where(xn >= 0, xn, 0.01 * xn)


@jax.jit
def _impl(h, ag, edge_index, ef, W_orin, b_orin, W_adju, b_adju, W_self,
          W_neigh, b_sage):
    h2 = h.reshape(N, U)
    bo = b_orin.reshape(1, U)
    ba = b_adju.reshape(1, U)
    bs = b_sage.reshape(1, U)

    hgeo, hsplit, hadj = pl.pallas_call(
        _prep_body,
        out_shape=[jax.ShapeDtypeStruct((N, U), jnp.float32),
                   jax.ShapeDtypeStruct((NC, N, U2), jnp.float32),
                   jax.ShapeDtypeStruct((N, U), jnp.float32)],
    )(h2, W_orin, bo, W_adju, ba)

    npad = EPAD - E
    src = jnp.concatenate([edge_index[0],
                           jnp.zeros((npad,), jnp.int32)]).reshape(NS, NCH, CH)
    dst = jnp.concatenate([edge_index[1],
                           jnp.full((npad,), N, jnp.int32)]).reshape(NS, NCH, CH)
    efb = lax.bitcast_convert_type(
        jnp.concatenate([ef, jnp.zeros((npad,), jnp.float32)]),
        jnp.int32).reshape(NS, NCH, CH)
    pk = jnp.stack([src, dst, efb], axis=2)

    sc_fn = pl.kernel(
        _sc_body,
        out_type=[jax.ShapeDtypeStruct((NC, N, U2), jnp.float32),
                  jax.ShapeDtypeStruct((NC, N, 16), jnp.float32)],
        mesh=plsc.VectorSubcoreMesh(core_axis_name="c", subcore_axis_name="s"),
        scratch_types=(
            [pltpu.VMEM((3, CH), jnp.int32)] * 4 +
            [pltpu.VMEM((CH, U2), jnp.float32)] * 3 +
            [pltpu.VMEM((CH, 16), jnp.float32),
             pltpu.VMEM((ZROWS, U2), jnp.float32),
             pltpu.VMEM((ZROWS, 16), jnp.float32),
             pltpu.VMEM_SHARED((NROWS, U2), jnp.float32),
             pltpu.VMEM_SHARED((NROWS, 16), jnp.float32)] +
            [pltpu.SemaphoreType.DMA] * 10
        ),
        compiler_params=pltpu.CompilerParams(needs_layout_passes=False,
                                             use_tc_tiling_on_sc=False),
    )
    pneigh, pdeg = sc_fn(hsplit, pk)

    acc = pl.pallas_call(
        _mm_body,
        grid=(GM,),
        in_specs=[
            pl.BlockSpec((BM, N), lambda m: (m, 0)),
            pl.BlockSpec((BM, U), lambda m: (m, 0)),
        ],
        out_specs=pl.BlockSpec((N, U), lambda m: (0, 0)),
        out_shape=jax.ShapeDtypeStruct((N, U), jnp.float32),
        compiler_params=pltpu.CompilerParams(
            dimension_semantics=("arbitrary",)),
    )(ag, hadj)

    out = pl.pallas_call(
        _epi_body,
        out_shape=jax.ShapeDtypeStruct((N, U), jnp.float32),
    )(acc, hgeo, pneigh, pdeg, W_self, W_neigh, bs)

    return out.reshape(N, 1, U)


def kernel(h, ag, edge_index, ef, W_orin, b_orin, W_adju, b_adju, W_self,
           W_neigh, b_sage):
    return _impl(h, ag, edge_index, ef, W_orin, b_orin, W_adju, b_adju,
                 W_self, W_neigh, b_sage)


# speedup vs baseline: 1.7585x; 1.7585x over previous
import jax
import jax.numpy as jnp
from jax import lax
from jax.experimental import pallas as pl
from jax.experimental.pallas import tpu as pltpu
from jax.experimental.pallas import tpu_sc as plsc

N = 10000
E = 320000
U = 128
U2 = U // 2
UW = U2 + 16

NC = 2
NS = 16
CH = 128
NCH = 158
EPT = CH * NCH
EPAD = NS * EPT
NROWS = N + 8
ZROWS = 200
NROWCH = N // ZROWS
MAXJ = (NROWCH + NS - 1) // NS


def _prep_body(h_ref, wo_ref, bo_ref, wa_ref, ba_ref, hgeo_ref, hsplit_ref,
               hadj_ref):
    h = h_ref[...]
    dn = (((1,), (1,)), ((), ()))
    hgeo = lax.dot_general(h, wo_ref[...], dn,
                           preferred_element_type=jnp.float32) + bo_ref[...]
    hgeo_ref[...] = hgeo
    onehot = jnp.where(
        lax.broadcasted_iota(jnp.int32, (N, 16), 1) == 0, 1.0, 0.0)
    hsplit_ref[0] = jnp.concatenate([hgeo[:, :U2], onehot], axis=1)
    hsplit_ref[1] = jnp.concatenate([hgeo[:, U2:], onehot], axis=1)
    hadj_ref[...] = lax.dot_general(h, wa_ref[...], dn,
                                    preferred_element_type=jnp.float32) + ba_ref[...]


def _sc_body(hsplit_hbm, pk_hbm, outn_hbm,
             pk0, pk1, rows0, rows1, zeros_v,
             neigh_sp,
             semi0, semi1, semg0, semg1, sems0, sems1):
    cid = lax.axis_index("c")
    sid = lax.axis_index("s")

    zvec = jnp.zeros((16,), jnp.float32)

    def _fill_zeros(i, carry):
        r = i // (UW // 16)
        f = i % (UW // 16)
        zeros_v[r, pl.ds(f * 16, 16)] = zvec
        return carry
    lax.fori_loop(0, ZROWS * (UW // 16), _fill_zeros, 0)


    for j in range(MAXJ):
        k = sid + NS * j

        @pl.when(k < NROWCH)
        def _():
            pltpu.sync_copy(zeros_v, neigh_sp.at[pl.ds(k * ZROWS, ZROWS)])
    plsc.subcore_barrier()

    table = hsplit_hbm.at[cid]
    me = pk_hbm.at[sid]

    def _iter(i, pkp, pkq, rowsp, rowsq, semiq, semgp, semgq, semsp, semsq):
        @pl.when(i > 0)
        def _():
            pltpu.make_async_copy(rowsq, neigh_sp.at[pkq.at[1]], semsq).wait()

        @pl.when(i + 1 < NCH)
        def _():
            pltpu.async_copy(me.at[i + 1], pkq, semiq)

        pltpu.make_async_copy(table.at[pkp.at[0]], rowsp, semgp).wait()

        def _mul(e, c2):
            efv = plsc.bitcast(
                plsc.load_gather(pkp, [jnp.full((16,), 2, jnp.int32),
                                       jnp.full((16,), e, jnp.int32)]),
                jnp.float32)
            for f in range(U2 // 16):
                sl = pl.ds(f * 16, 16)
                rowsp[e, sl] = rowsp[e, sl] * efv
            return c2
        lax.fori_loop(0, CH, _mul, 0)

        pltpu.make_async_copy(rowsp, neigh_sp.at[pkp.at[1]], semsp).start(add=True)

        @pl.when(i + 1 < NCH)
        def _():
            pltpu.make_async_copy(me.at[i + 1], pkq, semiq).wait()
            pltpu.async_copy(table.at[pkq.at[0]], rowsq, semgq)

    pltpu.async_copy(me.at[0], pk0, semi0)
    pltpu.make_async_copy(me.at[0], pk0, semi0).wait()
    pltpu.async_copy(table.at[pk0.at[0]], rows0, semg0)

    @pl.loop(0, NCH, step=2)
    def _(i):
        _iter(i, pk0, pk1, rows0, rows1, semi1, semg0, semg1, sems0, sems1)
        _iter(i + 1, pk1, pk0, rows1, rows0, semi0, semg1, semg0, sems1, sems0)

    pltpu.make_async_copy(rows1, neigh_sp.at[pk1.at[1]], sems1).wait()

    plsc.subcore_barrier()

    for j in range(MAXJ):
        k = sid + NS * j

        @pl.when(k < NROWCH)
        def _():
            r0 = k * ZROWS
            pltpu.sync_copy(neigh_sp.at[pl.ds(r0, ZROWS)], zeros_v)
            pltpu.sync_copy(zeros_v, outn_hbm.at[cid, pl.ds(r0, ZROWS)])


BM = 400
GM = N // BM


def _mm_body(ag_ref, hadj_ref, acc_ref):
    m = pl.program_id(0)

    @pl.when(m == 0)
    def _():
        acc_ref[...] = jnp.zeros_like(acc_ref)

    acc_ref[...] += lax.dot_general(
        ag_ref[...], hadj_ref[...], (((0,), (0,)), ((), ())),
        preferred_element_type=jnp.float32)


def _epi_body(acc_ref, hgeo_ref, pn_ref, ws_ref, wn_ref, bs_ref,
              out_ref):
    deg = pn_ref[0, :, U2:U2 + 1]
    neigh = jnp.concatenate([pn_ref[0, :, :U2], pn_ref[1, :, :U2]], axis=-1)
    neigh = neigh / jnp.maximum(deg, 1.0)
    dn = (((1,), (1,)), ((), ()))
    geo = (lax.dot_general(hgeo_ref[...], ws_ref[...], dn,
                           preferred_element_type=jnp.float32)
           + lax.dot_general(neigh, wn_ref[...], dn,
                             preferred_element_type=jnp.float32)
           + bs_ref[...])
    tot = geo + acc_ref[...]
    mean = jnp.mean(tot, axis=-1, keepdims=True)
    var = jnp.mean((tot - mean) * (tot - mean), axis=-1, keepdims=True)
    xn = (tot - mean) * lax.rsqrt(var + 1e-5)
    out_ref[...] = jnp.where(xn >= 0, xn, 0.01 * xn)


@jax.jit
def _impl(h, ag, edge_index, ef, W_orin, b_orin, W_adju, b_adju, W_self,
          W_neigh, b_sage):
    h2 = h.reshape(N, U)
    bo = b_orin.reshape(1, U)
    ba = b_adju.reshape(1, U)
    bs = b_sage.reshape(1, U)

    hgeo, hsplit, hadj = pl.pallas_call(
        _prep_body,
        out_shape=[jax.ShapeDtypeStruct((N, U), jnp.float32),
                   jax.ShapeDtypeStruct((NC, N, UW), jnp.float32),
                   jax.ShapeDtypeStruct((N, U), jnp.float32)],
    )(h2, W_orin, bo, W_adju, ba)

    npad = EPAD - E
    src = jnp.concatenate([edge_index[0],
                           jnp.zeros((npad,), jnp.int32)]).reshape(NS, NCH, CH)
    dst = jnp.concatenate([edge_index[1],
                           jnp.full((npad,), N, jnp.int32)]).reshape(NS, NCH, CH)
    efb = lax.bitcast_convert_type(
        jnp.concatenate([ef, jnp.zeros((npad,), jnp.float32)]),
        jnp.int32).reshape(NS, NCH, CH)
    pk = jnp.stack([src, dst, efb], axis=2)

    sc_fn = pl.kernel(
        _sc_body,
        out_type=jax.ShapeDtypeStruct((NC, N, UW), jnp.float32),
        mesh=plsc.VectorSubcoreMesh(core_axis_name="c", subcore_axis_name="s"),
        scratch_types=[
            pltpu.VMEM((3, CH), jnp.int32),
            pltpu.VMEM((3, CH), jnp.int32),
            pltpu.VMEM((CH, UW), jnp.float32),
            pltpu.VMEM((CH, UW), jnp.float32),
            pltpu.VMEM((ZROWS, UW), jnp.float32),
            pltpu.VMEM_SHARED((NROWS, UW), jnp.float32),
            pltpu.SemaphoreType.DMA,
            pltpu.SemaphoreType.DMA,
            pltpu.SemaphoreType.DMA,
            pltpu.SemaphoreType.DMA,
            pltpu.SemaphoreType.DMA,
            pltpu.SemaphoreType.DMA,
        ],
        compiler_params=pltpu.CompilerParams(needs_layout_passes=False,
                                             use_tc_tiling_on_sc=False),
    )
    pneigh = sc_fn(hsplit, pk)

    acc = pl.pallas_call(
        _mm_body,
        grid=(GM,),
        in_specs=[
            pl.BlockSpec((BM, N), lambda m: (m, 0)),
            pl.BlockSpec((BM, U), lambda m: (m, 0)),
        ],
        out_specs=pl.BlockSpec((N, U), lambda m: (0, 0)),
        out_shape=jax.ShapeDtypeStruct((N, U), jnp.float32),
        compiler_params=pltpu.CompilerParams(
            dimension_semantics=("arbitrary",)),
    )(ag, hadj)

    out = pl.pallas_call(
        _epi_body,
        out_shape=jax.ShapeDtypeStruct((N, U), jnp.float32),
    )(acc, hgeo, pneigh, W_self, W_neigh, bs)

    return out.reshape(N, 1, U)


def kernel(h, ag, edge_index, ef, W_orin, b_orin, W_adju, b_adju, W_self,
           W_neigh, b_sage):
    return _impl(h, ag, edge_index, ef, W_orin, b_orin, W_adju, b_adju,
                 W_self, W_neigh, b_sage)


# speedup vs baseline: 2.0269x; 1.1526x over previous
import jax
import jax.numpy as jnp
from jax import lax
from jax.experimental import pallas as pl
from jax.experimental.pallas import tpu as pltpu
from jax.experimental.pallas import tpu_sc as plsc

N = 10000
E = 320000
U = 128
U2 = U // 2

NC = 2
NS = 16
CH = 128
NCH = 158
EPT = CH * NCH
EPAD = NS * EPT
NROWS = N + 8
ZROWS = 200
NROWCH = N // ZROWS
MAXJ = (NROWCH + NS - 1) // NS


def _prep_body(h_ref, wo_ref, bo_ref, wa_ref, ba_ref, hgeo_ref, hsplit_ref,
               hadj_ref):
    h = h_ref[...]
    dn = (((1,), (1,)), ((), ()))
    hgeo = lax.dot_general(h, wo_ref[...], dn,
                           preferred_element_type=jnp.float32) + bo_ref[...]
    hgeo_ref[...] = hgeo
    hsplit_ref[0] = hgeo[:, :U2]
    hsplit_ref[1] = hgeo[:, U2:]
    hadj_ref[...] = lax.dot_general(h, wa_ref[...], dn,
                                    preferred_element_type=jnp.float32) + ba_ref[...]


def _sc_body(hsplit_hbm, pk_hbm, outn_hbm, outd_hbm,
             pk0, pk1, rows0, rows1, ones_v, zeros_v,
             zeros16_v, neigh_sp, deg_sp,
             semi0, semi1, semg0, semg1, sems0, sems1):
    cid = lax.axis_index("c")
    sid = lax.axis_index("s")

    zvec = jnp.zeros((16,), jnp.float32)
    onehot = jnp.where(lax.iota(jnp.int32, 16) == 0, 1.0, 0.0).astype(jnp.float32)

    def _fill_zeros(i, carry):
        r = i // (U2 // 16)
        f = i % (U2 // 16)
        zeros_v[r, pl.ds(f * 16, 16)] = zvec
        return carry
    lax.fori_loop(0, ZROWS * (U2 // 16), _fill_zeros, 0)

    def _fill_zeros16(i, carry):
        zeros16_v[i, :] = zvec
        return carry
    lax.fori_loop(0, ZROWS, _fill_zeros16, 0)

    def _fill_ones(i, carry):
        ones_v[i, :] = onehot
        return carry
    lax.fori_loop(0, CH, _fill_ones, 0)

    for j in range(MAXJ):
        k = sid + NS * j

        @pl.when(k < NROWCH)
        def _():
            pltpu.sync_copy(zeros_v, neigh_sp.at[pl.ds(k * ZROWS, ZROWS)])
            pltpu.sync_copy(zeros16_v, deg_sp.at[pl.ds(k * ZROWS, ZROWS)])
    plsc.subcore_barrier()

    table = hsplit_hbm.at[cid]
    me = pk_hbm.at[sid]

    def _iter(i, pkp, pkq, rowsp, rowsq, semiq, semgp, semgq, semsp, semsq):
        @pl.when(i > 0)
        def _():
            pltpu.make_async_copy(rowsq, neigh_sp.at[pkq.at[1]], semsq).wait()
            pltpu.make_async_copy(ones_v, deg_sp.at[pkq.at[1]], semsq).wait()

        @pl.when(i + 1 < NCH)
        def _():
            pltpu.async_copy(me.at[i + 1], pkq, semiq)

        pltpu.make_async_copy(table.at[pkp.at[0]], rowsp, semgp).wait()

        @pl.when(i + 1 < NCH)
        def _():
            pltpu.make_async_copy(me.at[i + 1], pkq, semiq).wait()
            pltpu.async_copy(table.at[pkq.at[0]], rowsq, semgq)

        def _mul(e, c2):
            efv = plsc.bitcast(
                plsc.load_gather(pkp, [jnp.full((16,), 2, jnp.int32),
                                       jnp.full((16,), e, jnp.int32)]),
                jnp.float32)
            for f in range(U2 // 16):
                sl = pl.ds(f * 16, 16)
                rowsp[e, sl] = rowsp[e, sl] * efv
            return c2
        lax.fori_loop(0, CH, _mul, 0)

        pltpu.make_async_copy(rowsp, neigh_sp.at[pkp.at[1]], semsp).start(add=True)
        pltpu.make_async_copy(ones_v, deg_sp.at[pkp.at[1]], semsp).start(add=True)

    pltpu.async_copy(me.at[0], pk0, semi0)
    pltpu.make_async_copy(me.at[0], pk0, semi0).wait()
    pltpu.async_copy(table.at[pk0.at[0]], rows0, semg0)

    @pl.loop(0, NCH, step=2)
    def _(i):
        _iter(i, pk0, pk1, rows0, rows1, semi1, semg0, semg1, sems0, sems1)
        _iter(i + 1, pk1, pk0, rows1, rows0, semi0, semg1, semg0, sems1, sems0)

    pltpu.make_async_copy(rows1, neigh_sp.at[pk1.at[1]], sems1).wait()
    pltpu.make_async_copy(ones_v, deg_sp.at[pk1.at[1]], sems1).wait()

    plsc.subcore_barrier()

    for j in range(MAXJ):
        k = sid + NS * j

        @pl.when(k < NROWCH)
        def _():
            r0 = k * ZROWS
            pltpu.sync_copy(neigh_sp.at[pl.ds(r0, ZROWS)], zeros_v)
            pltpu.sync_copy(zeros_v, outn_hbm.at[cid, pl.ds(r0, ZROWS)])
            pltpu.sync_copy(deg_sp.at[pl.ds(r0, ZROWS)], zeros16_v)
            pltpu.sync_copy(zeros16_v, outd_hbm.at[cid, pl.ds(r0, ZROWS)])


BM = 400
GM = N // BM


def _mm_body(ag_ref, hadj_ref, acc_ref):
    m = pl.program_id(0)

    @pl.when(m == 0)
    def _():
        acc_ref[...] = jnp.zeros_like(acc_ref)

    acc_ref[...] += lax.dot_general(
        ag_ref[...], hadj_ref[...], (((0,), (0,)), ((), ())),
        preferred_element_type=jnp.float32)


def _epi_body(acc_ref, hgeo_ref, pn_ref, pd_ref, ws_ref, wn_ref, bs_ref,
              out_ref):
    deg = pd_ref[0, :, 0:1]
    neigh = jnp.concatenate([pn_ref[0], pn_ref[1]], axis=-1)
    neigh = neigh / jnp.maximum(deg, 1.0)
    dn = (((1,), (1,)), ((), ()))
    geo = (lax.dot_general(hgeo_ref[...], ws_ref[...], dn,
                           preferred_element_type=jnp.float32)
           + lax.dot_general(neigh, wn_ref[...], dn,
                             preferred_element_type=jnp.float32)
           + bs_ref[...])
    tot = geo + acc_ref[...]
    mean = jnp.mean(tot, axis=-1, keepdims=True)
    var = jnp.mean((tot - mean) * (tot - mean), axis=-1, keepdims=True)
    xn = (tot - mean) * lax.rsqrt(var + 1e-5)
    out_ref[...] = jnp.where(xn >= 0, xn, 0.01 * xn)


@jax.jit
def _impl(h, ag, edge_index, ef, W_orin, b_orin, W_adju, b_adju, W_self,
          W_neigh, b_sage):
    h2 = h.reshape(N, U)
    bo = b_orin.reshape(1, U)
    ba = b_adju.reshape(1, U)
    bs = b_sage.reshape(1, U)

    hgeo, hsplit, hadj = pl.pallas_call(
        _prep_body,
        out_shape=[jax.ShapeDtypeStruct((N, U), jnp.float32),
                   jax.ShapeDtypeStruct((NC, N, U2), jnp.float32),
                   jax.ShapeDtypeStruct((N, U), jnp.float32)],
    )(h2, W_orin, bo, W_adju, ba)

    npad = EPAD - E
    src = jnp.concatenate([edge_index[0],
                           jnp.zeros((npad,), jnp.int32)]).reshape(NS, NCH, CH)
    dst = jnp.concatenate([edge_index[1],
                           jnp.full((npad,), N, jnp.int32)]).reshape(NS, NCH, CH)
    efb = lax.bitcast_convert_type(
        jnp.concatenate([ef, jnp.zeros((npad,), jnp.float32)]),
        jnp.int32).reshape(NS, NCH, CH)
    pk = jnp.stack([src, dst, efb], axis=2)

    sc_fn = pl.kernel(
        _sc_body,
        out_type=[jax.ShapeDtypeStruct((NC, N, U2), jnp.float32),
                  jax.ShapeDtypeStruct((NC, N, 16), jnp.float32)],
        mesh=plsc.VectorSubcoreMesh(core_axis_name="c", subcore_axis_name="s"),
        scratch_types=[
            pltpu.VMEM((3, CH), jnp.int32),
            pltpu.VMEM((3, CH), jnp.int32),
            pltpu.VMEM((CH, U2), jnp.float32),
            pltpu.VMEM((CH, U2), jnp.float32),
            pltpu.VMEM((CH, 16), jnp.float32),
            pltpu.VMEM((ZROWS, U2), jnp.float32),
            pltpu.VMEM((ZROWS, 16), jnp.float32),
            pltpu.VMEM_SHARED((NROWS, U2), jnp.float32),
            pltpu.VMEM_SHARED((NROWS, 16), jnp.float32),
            pltpu.SemaphoreType.DMA,
            pltpu.SemaphoreType.DMA,
            pltpu.SemaphoreType.DMA,
            pltpu.SemaphoreType.DMA,
            pltpu.SemaphoreType.DMA,
            pltpu.SemaphoreType.DMA,
        ],
        compiler_params=pltpu.CompilerParams(needs_layout_passes=False,
                                             use_tc_tiling_on_sc=False),
    )
    pneigh, pdeg = sc_fn(hsplit, pk)

    acc = pl.pallas_call(
        _mm_body,
        grid=(GM,),
        in_specs=[
            pl.BlockSpec((BM, N), lambda m: (m, 0)),
            pl.BlockSpec((BM, U), lambda m: (m, 0)),
        ],
        out_specs=pl.BlockSpec((N, U), lambda m: (0, 0)),
        out_shape=jax.ShapeDtypeStruct((N, U), jnp.float32),
        compiler_params=pltpu.CompilerParams(
            dimension_semantics=("arbitrary",)),
    )(ag, hadj)

    out = pl.pallas_call(
        _epi_body,
        out_shape=jax.ShapeDtypeStruct((N, U), jnp.float32),
    )(acc, hgeo, pneigh, pdeg, W_self, W_neigh, bs)

    return out.reshape(N, 1, U)


def kernel(h, ag, edge_index, ef, W_orin, b_orin, W_adju, b_adju, W_self,
           W_neigh, b_sage):
    return _impl(h, ag, edge_index, ef, W_orin, b_orin, W_adju, b_adju,
                 W_self, W_neigh, b_sage)
